# baseline (speedup 1.0000x reference)
import functools

import jax
import jax.numpy as jnp
from jax import lax
from jax.experimental import pallas as pl
from jax.experimental.pallas import tpu as pltpu
from jax.experimental.pallas import tpu_sc as plsc

NUM_EXPERTS = 16
TOP_K = 2
INPUT_DIM = 2048
TILE = 2048
SEG = 512

_NC, _NS, _L = 2, 16, 16
_NW = _NC * _NS


def _logits_body(x_ref, w_ref, b_ref, lout_ref):
    lt = lax.dot_general(w_ref[...], x_ref[...],
                         (((0,), (1,)), ((), ())),
                         preferred_element_type=jnp.float32) + b_ref[...]
    for j in range(TILE // SEG):
        lout_ref[j] = lt[:, j * SEG:(j + 1) * SEG]


def _logits_kernel(x_flat, W, b):
    num_tokens = x_flat.shape[0]
    grid = num_tokens // TILE
    return pl.pallas_call(
        _logits_body,
        grid=(grid,),
        in_specs=[
            pl.BlockSpec((TILE, INPUT_DIM), lambda i: (i, 0)),
            pl.BlockSpec((INPUT_DIM, NUM_EXPERTS), lambda i: (0, 0)),
            pl.BlockSpec((NUM_EXPERTS, 1), lambda i: (0, 0)),
        ],
        out_specs=pl.BlockSpec((TILE // SEG, NUM_EXPERTS, SEG),
                               lambda i: (i, 0, 0)),
        out_shape=jax.ShapeDtypeStruct(
            (num_tokens // SEG, NUM_EXPERTS, SEG), jnp.float32),
    )(x_flat, W, b.reshape(NUM_EXPERTS, 1))


def _route_body(tpw, logits_hbm, w1_hbm, w2_hbm, i1_hbm, i2_hbm, parts_hbm,
                lt, w1o, w2o, i1o, i2o, acc):
    wid = lax.axis_index("s") * _NC + lax.axis_index("c")
    base = wid * tpw
    pltpu.sync_copy(logits_hbm.at[wid], lt)

    zf = jnp.zeros((_L,), jnp.float32)
    for k in range(2):
        for e in range(NUM_EXPERTS):
            acc[k, e] = zf

    for g in range(tpw // _L):
        sl = pl.ds(g * _L, _L)
        vs = [lt[e, sl] for e in range(NUM_EXPERTS)]
        m1 = vs[0]
        i1v = jnp.zeros((_L,), jnp.int32)
        m2 = jnp.full((_L,), -jnp.inf, jnp.float32)
        i2v = jnp.zeros((_L,), jnp.int32)
        for e in range(1, NUM_EXPERTS):
            v = vs[e]
            ev = jnp.full((_L,), e, jnp.int32)
            gt1 = v > m1
            gt2 = v > m2
            m2 = jnp.where(gt1, m1, jnp.where(gt2, v, m2))
            i2v = jnp.where(gt1, i1v, jnp.where(gt2, ev, i2v))
            m1 = jnp.where(gt1, v, m1)
            i1v = jnp.where(gt1, ev, i1v)
        ts = [jnp.exp(vs[e] - m1) for e in range(NUM_EXPERTS)]
        s = ts[0]
        for e in range(1, NUM_EXPERTS):
            s = s + ts[e]
        inv = 1.0 / s
        w2v = jnp.exp(m2 - m1) * inv
        w1o[sl] = inv
        w2o[sl] = w2v
        i1o[sl] = i1v
        i2o[sl] = i2v
        for e in range(NUM_EXPERTS):
            ev = jnp.full((_L,), e, jnp.int32)
            ce = (jnp.where(i1v == ev, 1.0, 0.0)
                  + jnp.where(i2v == ev, 1.0, 0.0))
            acc[0, e] = acc[0, e] + ce
            acc[1, e] = acc[1, e] + ts[e] * inv

    pltpu.sync_copy(w1o, w1_hbm.at[pl.ds(base, tpw)])
    pltpu.sync_copy(w2o, w2_hbm.at[pl.ds(base, tpw)])
    pltpu.sync_copy(i1o, i1_hbm.at[pl.ds(base, tpw)])
    pltpu.sync_copy(i2o, i2_hbm.at[pl.ds(base, tpw)])
    pltpu.sync_copy(acc, parts_hbm.at[wid])


def _route_kernel(logits):
    num_tokens = logits.shape[0] * logits.shape[2]
    tpw = num_tokens // _NW
    mesh = plsc.VectorSubcoreMesh(core_axis_name="c", subcore_axis_name="s")
    f = pl.kernel(
        functools.partial(_route_body, tpw),
        out_type=[
            jax.ShapeDtypeStruct((num_tokens,), jnp.float32),
            jax.ShapeDtypeStruct((num_tokens,), jnp.float32),
            jax.ShapeDtypeStruct((num_tokens,), jnp.int32),
            jax.ShapeDtypeStruct((num_tokens,), jnp.int32),
            jax.ShapeDtypeStruct((_NW, 2, NUM_EXPERTS, _L), jnp.float32),
        ],
        mesh=mesh,
        scratch_types=[
            pltpu.VMEM((NUM_EXPERTS, tpw), jnp.float32),
            pltpu.VMEM((tpw,), jnp.float32),
            pltpu.VMEM((tpw,), jnp.float32),
            pltpu.VMEM((tpw,), jnp.int32),
            pltpu.VMEM((tpw,), jnp.int32),
            pltpu.VMEM((2, NUM_EXPERTS, _L), jnp.float32),
        ],
    )
    return f(logits)


def _aux_body(inv_n2, parts_ref, aux_ref):
    p = parts_ref[...]
    c2 = jnp.sum(p[:, 0], axis=0)
    s2 = jnp.sum(p[:, 1], axis=0)
    ce = jnp.sum(c2, axis=1, keepdims=True)
    se = jnp.sum(s2, axis=1, keepdims=True)
    aux_ref[...] = (NUM_EXPERTS * inv_n2
                    * jnp.sum(ce * se, keepdims=True))


def _aux_kernel(parts, num_tokens):
    inv_n2 = 1.0 / (float(num_tokens) * float(num_tokens))
    return pl.pallas_call(
        functools.partial(_aux_body, inv_n2),
        out_shape=jax.ShapeDtypeStruct((1, 1), jnp.float32),
    )(parts)


def kernel(x, W, b):
    num_tokens = x.shape[0] * x.shape[1]
    x_flat = x.reshape(num_tokens, INPUT_DIM)
    logits = _logits_kernel(x_flat, W, b)
    w1, w2, i1, i2, parts = _route_kernel(logits)
    aux = _aux_kernel(parts, num_tokens)
    weights = jnp.stack([w1, w2], axis=1)
    indices = jnp.stack([i1, i2], axis=1)
    return weights, indices, aux[0, 0]
